# Initial kernel scaffold; baseline (speedup 1.0000x reference)
import jax
import jax.numpy as jnp
from jax.experimental import pallas as pl


def kernel(x, sadj, fadj, batch, s1_W1, s1_b1, s1_W2, s1_b2, s1_W3, s1_b3, s1_p1, s1_p2, s2_W1, s2_b1, s2_W2, s2_b2, s2_W3, s2_b3, s2_p1, s2_p2, c_W1, c_b1, c_W2, c_b2, c_W3, c_b3, c_p1, c_p2, att_W1, att_b1, att_W2):
    raise NotImplementedError("write your pallas kernel here")


# speedup vs baseline: 2.3312x; 2.3312x over previous
import functools
import numpy as np
import jax
import jax.numpy as jnp
from jax import lax
from jax.experimental import pallas as pl
from jax.experimental.pallas import tpu as pltpu
from jax.experimental.pallas import tpu_sc as plsc

N = 10000
F = 128
NB = 128
E = 320000
NTILES = 32
CHUNK = 128
CHUNKS = 80
EPAD = NTILES * CHUNKS * CHUNK
ACC_ROWS = 10240
DUST = N
K1 = 5000
K2 = 2500

_mesh = plsc.VectorSubcoreMesh(core_axis_name="c", subcore_axis_name="s")


@functools.partial(
    pl.kernel,
    mesh=_mesh,
    out_type=jax.ShapeDtypeStruct((2, ACC_ROWS, F), jnp.float32),
    scratch_types=[
        pltpu.VMEM((CHUNKS, CHUNK), jnp.int32),
        pltpu.VMEM((CHUNKS, CHUNK), jnp.int32),
        pltpu.VMEM((CHUNK, F), jnp.float32),
        pltpu.VMEM_SHARED((ACC_ROWS, F), jnp.float32),
        pltpu.SemaphoreType.DMA,
    ],
)
def _segsum_k(v_hbm, src_hbm, dst_hbm, zero_hbm, out_hbm,
              src_v, dst_v, rows_v, acc_sh, sem):
    cid = lax.axis_index("c")
    sid = lax.axis_index("s")
    wid = cid * 16 + sid
    rows_per_tile = ACC_ROWS // 16
    tile_rows = pl.ds(sid * rows_per_tile, rows_per_tile)
    pltpu.sync_copy(zero_hbm.at[tile_rows], acc_sh.at[tile_rows])
    pltpu.sync_copy(src_hbm.at[pl.ds(wid * CHUNKS, CHUNKS)], src_v)
    pltpu.sync_copy(dst_hbm.at[pl.ds(wid * CHUNKS, CHUNKS)], dst_v)
    plsc.subcore_barrier()

    def body(j, carry):
        pltpu.async_copy(v_hbm.at[src_v.at[j]], rows_v, sem).wait()
        pltpu.sync_copy(rows_v, acc_sh.at[dst_v.at[j]], add=True)
        return carry

    lax.fori_loop(0, CHUNKS, body, 0)
    plsc.subcore_barrier()
    pltpu.sync_copy(acc_sh.at[tile_rows], out_hbm.at[cid].at[tile_rows])


def _segment_sum(v, src2d, dst2d, zero):
    out = _segsum_k(v, src2d, dst2d, zero)
    return out[0, :N] + out[1, :N]


def _pad_idx(idx, fill):
    pad = EPAD - idx.shape[0]
    return jnp.concatenate(
        [idx, jnp.full((pad,), fill, jnp.int32)]).reshape(EPAD // CHUNK, CHUNK)


def _topk_mask(s, k):
    t = jnp.sort(s)[::-1][k - 1]
    gt = s > t
    need = k - jnp.sum(gt.astype(jnp.int32))
    eq = s == t
    eqrank = jnp.cumsum(eq.astype(jnp.int32)) - eq.astype(jnp.int32)
    kept = gt | (eq & (eqrank < need))
    a = jnp.argmax(s)
    iota = jnp.arange(s.shape[0])
    z = jnp.max(jnp.where(kept & eq, iota, -1))
    return kept, a, z


def _stats(v, kept, batch):
    vm = jnp.where(kept[:, None], v, -jnp.inf)
    m = jax.ops.segment_max(vm, batch, num_segments=NB)
    gmp = jnp.where(jnp.isfinite(m), m, 0.0)
    vs = v * kept[:, None].astype(v.dtype)
    ssum = jax.ops.segment_sum(vs, batch, num_segments=NB)
    cnt = jax.ops.segment_sum(kept.astype(jnp.float32), batch, num_segments=NB)
    gap = ssum / jnp.maximum(cnt, 1.0)[:, None]
    return jnp.concatenate([gmp, gap], axis=1)


def _forward(x, src2d, dst2d, srcflat, dstflat, zero, batch, prm):
    W1, b1, W2, b2, W3, b3, p1, p2 = prm
    relu = jax.nn.relu
    h1 = relu(_segment_sum(x @ W1, src2d, dst2d, zero) + b1)
    s1 = (h1 @ p1) / (jnp.linalg.norm(p1) + 1e-16)
    kept1, a1, z1 = _topk_mask(s1, K1)
    y1 = h1 * jnp.tanh(s1)[:, None] * kept1[:, None]
    x1 = _stats(y1, kept1, batch)
    cnt_valid1 = jnp.sum((kept1[srcflat] & kept1[dstflat]).astype(jnp.int32))
    h2 = relu(_segment_sum(y1 @ W2, src2d, dst2d, zero) + b2)
    s2raw = (h2 @ p2) / (jnp.linalg.norm(p2) + 1e-16)
    s2 = jnp.where(kept1, s2raw, -jnp.inf)
    kept2, _, _ = _topk_mask(s2, K2)
    y2 = h2 * jnp.tanh(s2raw)[:, None] * kept2[:, None]
    x2 = _stats(y2, kept2, batch)
    supp3 = y2 @ W3
    A3 = _segment_sum(supp3, src2d, dst2d, zero)
    cnt_inv1 = E - cnt_valid1
    coef = jnp.where(kept2[a1] & kept2[z1], cnt_inv1.astype(jnp.float32), 0.0)
    A3 = A3.at[z1].add(coef * supp3[a1])
    h3 = relu(A3 + b3)
    x3 = _stats(h3, kept2, batch)
    return x1 + x2 + x3


def kernel(x, sadj, fadj, batch,
           s1_W1, s1_b1, s1_W2, s1_b2, s1_W3, s1_b3, s1_p1, s1_p2,
           s2_W1, s2_b1, s2_W2, s2_b2, s2_W3, s2_b3, s2_p1, s2_p2,
           c_W1, c_b1, c_W2, c_b2, c_W3, c_b3, c_p1, c_p2,
           att_W1, att_b1, att_W2):
    s1p = (s1_W1, s1_b1, s1_W2, s1_b2, s1_W3, s1_b3, s1_p1, s1_p2)
    s2p = (s2_W1, s2_b1, s2_W2, s2_b2, s2_W3, s2_b3, s2_p1, s2_p2)
    cp = (c_W1, c_b1, c_W2, c_b2, c_W3, c_b3, c_p1, c_p2)
    zero = jnp.zeros((ACC_ROWS, F), jnp.float32)
    ssrc2d = _pad_idx(sadj[0], 0)
    sdst2d = _pad_idx(sadj[1], DUST)
    fsrc2d = _pad_idx(fadj[0], 0)
    fdst2d = _pad_idx(fadj[1], DUST)
    emb1 = _forward(x, ssrc2d, sdst2d, sadj[0], sadj[1], zero, batch, s1p)
    com1 = _forward(x, ssrc2d, sdst2d, sadj[0], sadj[1], zero, batch, cp)
    com2 = _forward(x, fsrc2d, fdst2d, fadj[0], fadj[1], zero, batch, cp)
    emb2 = _forward(x, fsrc2d, fdst2d, fadj[0], fadj[1], zero, batch, s2p)
    Xcom = (com1 + com2) / 2.0
    z = jnp.stack([emb1, emb2, Xcom], axis=1)
    w = jnp.tanh(z @ att_W1 + att_b1) @ att_W2
    beta = jax.nn.softmax(w, axis=1)
    emb = (beta * z).sum(axis=1)
    return (emb, beta, emb1, com1, com2, emb2, emb)
